# initial kernel scaffold (unmeasured)
import functools

import jax
import jax.numpy as jnp
from jax import lax
from jax.experimental import pallas as pl
from jax.experimental.pallas import tpu as pltpu

N_DEV = 8


def kernel(x, w_mat):
    m_per, k = x.shape
    _, n_loc = w_mat.shape

    def body(x_ref, w_ref, out_ref, comm_ref, amax_src, amax_gather,
             send_sems, recv_sems, amax_send_sems, amax_recv_sems):
        my = lax.axis_index("i")
        left = lax.rem(my + N_DEV - 1, N_DEV)
        right = lax.rem(my + 1, N_DEV)

        barrier_sem = pltpu.get_barrier_semaphore()
        for nbr in (left, right):
            pl.semaphore_signal(barrier_sem, inc=1, device_id=(nbr,),
                                device_id_type=pl.DeviceIdType.MESH)
        pl.semaphore_wait(barrier_sem, 2)

        comm_ref[0] = x_ref[...]

        def chunk_gemm(chunk, origin, amax):
            y = jnp.dot(chunk, w_ref[...], preferred_element_type=jnp.float32)
            y = jnp.maximum(y, 0.0)
            out_ref[pl.ds(origin * m_per, m_per), :] = y
            return jnp.maximum(amax, jnp.max(y))

        amax = jnp.float32(0.0)
        for h in range(N_DEV - 1):
            s, r = h % 2, (h + 1) % 2
            rdma = pltpu.make_async_remote_copy(
                src_ref=comm_ref.at[s],
                dst_ref=comm_ref.at[r],
                send_sem=send_sems.at[s],
                recv_sem=recv_sems.at[r],
                device_id=(right,),
                device_id_type=pl.DeviceIdType.MESH,
            )
            rdma.start()
            amax = chunk_gemm(comm_ref[s], lax.rem(my + N_DEV - h, N_DEV)
                              if h > 0 else my, amax)
            rdma.wait()
        amax = chunk_gemm(comm_ref[(N_DEV - 1) % 2],
                          lax.rem(my + 1, N_DEV), amax)

        amax_src[...] = jnp.full((8, 128), amax, jnp.float32)
        amax_gather[0] = amax_src[...]
        amax_rdmas = []
        for d in range(1, N_DEV):
            tgt = lax.rem(my + d, N_DEV)
            r = pltpu.make_async_remote_copy(
                src_ref=amax_src,
                dst_ref=amax_gather.at[d],
                send_sem=amax_send_sems.at[d - 1],
                recv_sem=amax_recv_sems.at[d],
                device_id=(tgt,),
                device_id_type=pl.DeviceIdType.MESH,
            )
            r.start()
            amax_rdmas.append(r)
        for r in amax_rdmas:
            r.wait_recv()
        for r in amax_rdmas:
            r.wait_send()
        global_amax = jnp.max(amax_gather[...])

        scale = global_amax / 448.0
        inv = 448.0 / global_amax
        q = (out_ref[...] * inv).astype(jnp.float8_e4m3fn)
        out_ref[...] = q.astype(jnp.float32) * scale

        @functools.partial(pl.run_scoped, ack=pltpu.SemaphoreType.REGULAR)
        def _(ack):
            for d in range(1, N_DEV):
                pl.semaphore_signal(ack, inc=1,
                                    device_id=(lax.rem(my + d, N_DEV),),
                                    device_id_type=pl.DeviceIdType.MESH)
            pl.semaphore_wait(ack, N_DEV - 1)

    return pl.pallas_call(
        body,
        out_shape=jax.ShapeDtypeStruct((N_DEV * m_per, n_loc), jnp.float32),
        in_specs=[
            pl.BlockSpec(memory_space=pltpu.VMEM),
            pl.BlockSpec(memory_space=pltpu.VMEM),
        ],
        out_specs=pl.BlockSpec(memory_space=pltpu.VMEM),
        scratch_shapes=[
            pltpu.VMEM((2, m_per, k), x.dtype),
            pltpu.VMEM((8, 128), jnp.float32),
            pltpu.VMEM((N_DEV, 8, 128), jnp.float32),
            pltpu.SemaphoreType.DMA((2,)),
            pltpu.SemaphoreType.DMA((2,)),
            pltpu.SemaphoreType.DMA((N_DEV - 1,)),
            pltpu.SemaphoreType.DMA((N_DEV,)),
        ],
        compiler_params=pltpu.CompilerParams(collective_id=0),
    )(x, w_mat)


# baseline (device time: 382273 ns/iter reference)
import functools

import jax
import jax.numpy as jnp
from jax import lax
from jax.experimental import pallas as pl
from jax.experimental.pallas import tpu as pltpu

N_DEV = 8


def kernel(x, w_mat):
    x = x.astype(jnp.bfloat16)
    w_mat = w_mat.astype(jnp.bfloat16)
    m_per, k = x.shape
    _, n_loc = w_mat.shape

    def body(x_ref, w_ref, out_ref, comm_ref, amax_src, amax_gather,
             send_sems, recv_sems, amax_send_sems, amax_recv_sems):
        my = lax.axis_index("i")
        left = lax.rem(my + N_DEV - 1, N_DEV)
        right = lax.rem(my + 1, N_DEV)

        barrier_sem = pltpu.get_barrier_semaphore()
        for nbr in (left, right):
            pl.semaphore_signal(barrier_sem, inc=1, device_id=(nbr,),
                                device_id_type=pl.DeviceIdType.MESH)
        pl.semaphore_wait(barrier_sem, 2)

        comm_ref[0] = x_ref[...]

        def chunk_gemm(chunk, origin, amax):
            y = jnp.dot(chunk, w_ref[...], preferred_element_type=jnp.float32)
            y = jnp.maximum(y, 0.0)
            out_ref[pl.ds(origin * m_per, m_per), :] = y
            return jnp.maximum(amax, jnp.max(y))

        amax = jnp.float32(0.0)
        for h in range(N_DEV - 1):
            s, r = h % 2, (h + 1) % 2
            rdma = pltpu.make_async_remote_copy(
                src_ref=comm_ref.at[s],
                dst_ref=comm_ref.at[r],
                send_sem=send_sems.at[s],
                recv_sem=recv_sems.at[r],
                device_id=(right,),
                device_id_type=pl.DeviceIdType.MESH,
            )
            rdma.start()
            amax = chunk_gemm(comm_ref[s], lax.rem(my + N_DEV - h, N_DEV)
                              if h > 0 else my, amax)
            rdma.wait()
        amax = chunk_gemm(comm_ref[(N_DEV - 1) % 2],
                          lax.rem(my + 1, N_DEV), amax)

        amax_src[...] = jnp.full((8, 128), amax, jnp.float32)
        amax_gather[0] = amax_src[...]
        amax_rdmas = []
        for d in range(1, N_DEV):
            tgt = lax.rem(my + d, N_DEV)
            r = pltpu.make_async_remote_copy(
                src_ref=amax_src,
                dst_ref=amax_gather.at[d],
                send_sem=amax_send_sems.at[d - 1],
                recv_sem=amax_recv_sems.at[d],
                device_id=(tgt,),
                device_id_type=pl.DeviceIdType.MESH,
            )
            r.start()
            amax_rdmas.append(r)
        for r in amax_rdmas:
            r.wait_recv()
        for r in amax_rdmas:
            r.wait_send()
        global_amax = jnp.max(amax_gather[...])

        scale = global_amax / 448.0
        inv = 448.0 / global_amax
        q = (out_ref[...] * inv).astype(jnp.float8_e4m3fn)
        out_ref[...] = q.astype(jnp.float32) * scale

        @functools.partial(pl.run_scoped, ack=pltpu.SemaphoreType.REGULAR)
        def _(ack):
            for d in range(1, N_DEV):
                pl.semaphore_signal(ack, inc=1,
                                    device_id=(lax.rem(my + d, N_DEV),),
                                    device_id_type=pl.DeviceIdType.MESH)
            pl.semaphore_wait(ack, N_DEV - 1)

    return pl.pallas_call(
        body,
        out_shape=jax.ShapeDtypeStruct((N_DEV * m_per, n_loc), jnp.float32),
        in_specs=[
            pl.BlockSpec(memory_space=pltpu.VMEM),
            pl.BlockSpec(memory_space=pltpu.VMEM),
        ],
        out_specs=pl.BlockSpec(memory_space=pltpu.VMEM),
        scratch_shapes=[
            pltpu.VMEM((2, m_per, k), x.dtype),
            pltpu.VMEM((8, 128), jnp.float32),
            pltpu.VMEM((N_DEV, 8, 128), jnp.float32),
            pltpu.SemaphoreType.DMA((2,)),
            pltpu.SemaphoreType.DMA((2,)),
            pltpu.SemaphoreType.DMA((N_DEV - 1,)),
            pltpu.SemaphoreType.DMA((N_DEV,)),
        ],
        compiler_params=pltpu.CompilerParams(
            collective_id=0, vmem_limit_bytes=64 * 1024 * 1024),
    )(x, w_mat)


# device time: 220037 ns/iter; 1.7373x vs baseline; 1.7373x over previous
import functools

import jax
import jax.numpy as jnp
from jax import lax
from jax.experimental import pallas as pl
from jax.experimental.pallas import tpu as pltpu

N_DEV = 8
N_HOPS = 4


def kernel(x, w_mat):
    x = x.astype(jnp.bfloat16)
    w_mat = w_mat.astype(jnp.bfloat16)
    m_per, k = x.shape
    _, n_loc = w_mat.shape
    half = m_per // 2

    def body(x_ref, w_ref, out_ref, cw_ref, ccw_ref, cw_half_ref,
             ccw_half_ref, amax_src, amax_gather,
             cw_send_sems, cw_recv_sems, ccw_send_sems, ccw_recv_sems,
             amax_send_sems, amax_recv_sems):
        my = lax.axis_index("i")
        left = lax.rem(my + N_DEV - 1, N_DEV)
        right = lax.rem(my + 1, N_DEV)

        barrier_sem = pltpu.get_barrier_semaphore()
        for nbr in (left, right):
            pl.semaphore_signal(barrier_sem, inc=1, device_id=(nbr,),
                                device_id_type=pl.DeviceIdType.MESH)
        pl.semaphore_wait(barrier_sem, 2)

        def copy(src, dst, ssem, rsem, tgt):
            return pltpu.make_async_remote_copy(
                src_ref=src, dst_ref=dst, send_sem=ssem, recv_sem=rsem,
                device_id=(tgt,), device_id_type=pl.DeviceIdType.MESH)

        def chunk_gemm(chunk, row0, nrows, amax):
            y = jnp.dot(chunk, w_ref[...], preferred_element_type=jnp.float32)
            y = jnp.maximum(y, 0.0)
            out_ref[pl.ds(row0, nrows), :] = y
            return jnp.maximum(amax, jnp.max(y))

        cw = [copy(x_ref, cw_ref.at[0], cw_send_sems.at[0],
                   cw_recv_sems.at[0], right)]
        ccw = [copy(x_ref, ccw_ref.at[0], ccw_send_sems.at[0],
                    ccw_recv_sems.at[0], left)]
        cw[0].start()
        ccw[0].start()

        amax = chunk_gemm(x_ref[...], my * m_per, m_per, jnp.float32(0.0))

        for h in range(N_HOPS - 1):
            cw[h].wait_recv()
            if h < N_HOPS - 2:
                nxt = copy(cw_ref.at[h], cw_ref.at[h + 1],
                           cw_send_sems.at[h + 1], cw_recv_sems.at[h + 1],
                           right)
            else:
                nxt = copy(cw_ref.at[h, pl.ds(0, half)],
                           cw_half_ref,
                           cw_send_sems.at[h + 1], cw_recv_sems.at[h + 1],
                           right)
            nxt.start()
            cw.append(nxt)

            ccw[h].wait_recv()
            if h < N_HOPS - 2:
                nxt = copy(ccw_ref.at[h], ccw_ref.at[h + 1],
                           ccw_send_sems.at[h + 1], ccw_recv_sems.at[h + 1],
                           left)
            else:
                nxt = copy(ccw_ref.at[h, pl.ds(half, half)],
                           ccw_half_ref,
                           ccw_send_sems.at[h + 1], ccw_recv_sems.at[h + 1],
                           left)
            nxt.start()
            ccw.append(nxt)

            cw_origin = lax.rem(my + N_DEV - (h + 1), N_DEV)
            ccw_origin = lax.rem(my + h + 1, N_DEV)
            amax = chunk_gemm(cw_ref[h], cw_origin * m_per, m_per, amax)
            amax = chunk_gemm(ccw_ref[h], ccw_origin * m_per, m_per, amax)

        far = lax.rem(my + N_DEV // 2, N_DEV)
        cw[N_HOPS - 1].wait_recv()
        amax = chunk_gemm(cw_half_ref[...], far * m_per, half, amax)
        ccw[N_HOPS - 1].wait_recv()
        amax = chunk_gemm(ccw_half_ref[...], far * m_per + half, half, amax)

        amax_src[...] = jnp.full((8, 128), amax, jnp.float32)
        amax_gather[0] = amax_src[...]
        amax_rdmas = []
        for d in range(1, N_DEV):
            r = copy(amax_src, amax_gather.at[d], amax_send_sems.at[d - 1],
                     amax_recv_sems.at[d], lax.rem(my + d, N_DEV))
            r.start()
            amax_rdmas.append(r)
        for r in amax_rdmas:
            r.wait_recv()
        global_amax = jnp.max(amax_gather[...])

        scale = global_amax / 448.0
        inv = 448.0 / global_amax
        for b in range(N_DEV):
            blk = out_ref[pl.ds(b * m_per, m_per), :]
            q = (blk * inv).astype(jnp.float8_e4m3fn)
            out_ref[pl.ds(b * m_per, m_per), :] = q.astype(jnp.float32) * scale

        for r in cw + ccw + amax_rdmas:
            r.wait_send()

        @functools.partial(pl.run_scoped, ack=pltpu.SemaphoreType.REGULAR)
        def _(ack):
            for d in range(1, N_DEV):
                pl.semaphore_signal(ack, inc=1,
                                    device_id=(lax.rem(my + d, N_DEV),),
                                    device_id_type=pl.DeviceIdType.MESH)
            pl.semaphore_wait(ack, N_DEV - 1)

    return pl.pallas_call(
        body,
        out_shape=jax.ShapeDtypeStruct((N_DEV * m_per, n_loc), jnp.float32),
        in_specs=[
            pl.BlockSpec(memory_space=pltpu.VMEM),
            pl.BlockSpec(memory_space=pltpu.VMEM),
        ],
        out_specs=pl.BlockSpec(memory_space=pltpu.VMEM),
        scratch_shapes=[
            pltpu.VMEM((N_HOPS - 1, m_per, k), x.dtype),
            pltpu.VMEM((N_HOPS - 1, m_per, k), x.dtype),
            pltpu.VMEM((half, k), x.dtype),
            pltpu.VMEM((half, k), x.dtype),
            pltpu.VMEM((8, 128), jnp.float32),
            pltpu.VMEM((N_DEV, 8, 128), jnp.float32),
            pltpu.SemaphoreType.DMA((N_HOPS,)),
            pltpu.SemaphoreType.DMA((N_HOPS,)),
            pltpu.SemaphoreType.DMA((N_HOPS,)),
            pltpu.SemaphoreType.DMA((N_HOPS,)),
            pltpu.SemaphoreType.DMA((N_DEV - 1,)),
            pltpu.SemaphoreType.DMA((N_DEV,)),
        ],
        compiler_params=pltpu.CompilerParams(
            collective_id=0, vmem_limit_bytes=64 * 1024 * 1024),
    )(x, w_mat)


# device time: 207146 ns/iter; 1.8454x vs baseline; 1.0622x over previous
import functools

import jax
import jax.numpy as jnp
from jax import lax
from jax.experimental import pallas as pl
from jax.experimental.pallas import tpu as pltpu

N_DEV = 8
N_HOPS = 4


def kernel(x, w_mat):
    x = x.astype(jnp.bfloat16)
    m_per, k = x.shape
    _, n_loc = w_mat.shape
    half = m_per // 2
    wcols = n_loc // 8

    def body(x_ref, w_hbm_ref, out_ref, cw_ref, ccw_ref, cw_half_ref,
             ccw_half_ref, w_ref, w_stage, amax_src, amax_gather,
             cw_send_sems, cw_recv_sems, ccw_send_sems, ccw_recv_sems,
             amax_send_sems, amax_recv_sems, w_sem):
        my = lax.axis_index("i")
        left = lax.rem(my + N_DEV - 1, N_DEV)
        right = lax.rem(my + 1, N_DEV)

        barrier_sem = pltpu.get_barrier_semaphore()
        for nbr in (left, right):
            pl.semaphore_signal(barrier_sem, inc=1, device_id=(nbr,),
                                device_id_type=pl.DeviceIdType.MESH)
        pl.semaphore_wait(barrier_sem, 2)

        def copy(src, dst, ssem, rsem, tgt):
            return pltpu.make_async_remote_copy(
                src_ref=src, dst_ref=dst, send_sem=ssem, recv_sem=rsem,
                device_id=(tgt,), device_id_type=pl.DeviceIdType.MESH)

        def chunk_gemm(chunk, row0, nrows, amax):
            y = jnp.dot(chunk, w_ref[...], preferred_element_type=jnp.float32)
            y = jnp.maximum(y, 0.0)
            out_ref[pl.ds(row0, nrows), :] = y
            return jnp.maximum(amax, jnp.max(y))

        cw = [copy(x_ref, cw_ref.at[0], cw_send_sems.at[0],
                   cw_recv_sems.at[0], right)]
        ccw = [copy(x_ref, ccw_ref.at[0], ccw_send_sems.at[0],
                    ccw_recv_sems.at[0], left)]
        cw[0].start()
        ccw[0].start()

        for t in range(n_loc // wcols):
            wdma = pltpu.make_async_copy(
                w_hbm_ref.at[:, pl.ds(t * wcols, wcols)], w_stage, w_sem)
            wdma.start()
            wdma.wait()
            rows = k // 4
            for r in range(4):
                w_ref[pl.ds(r * rows, rows), pl.ds(t * wcols, wcols)] = (
                    w_stage[pl.ds(r * rows, rows), :].astype(jnp.bfloat16))

        amax = chunk_gemm(x_ref[...], my * m_per, m_per, jnp.float32(0.0))

        for h in range(N_HOPS - 1):
            cw[h].wait_recv()
            if h < N_HOPS - 2:
                nxt = copy(cw_ref.at[h], cw_ref.at[h + 1],
                           cw_send_sems.at[h + 1], cw_recv_sems.at[h + 1],
                           right)
            else:
                nxt = copy(cw_ref.at[h, pl.ds(0, half)],
                           cw_half_ref,
                           cw_send_sems.at[h + 1], cw_recv_sems.at[h + 1],
                           right)
            nxt.start()
            cw.append(nxt)

            ccw[h].wait_recv()
            if h < N_HOPS - 2:
                nxt = copy(ccw_ref.at[h], ccw_ref.at[h + 1],
                           ccw_send_sems.at[h + 1], ccw_recv_sems.at[h + 1],
                           left)
            else:
                nxt = copy(ccw_ref.at[h, pl.ds(half, half)],
                           ccw_half_ref,
                           ccw_send_sems.at[h + 1], ccw_recv_sems.at[h + 1],
                           left)
            nxt.start()
            ccw.append(nxt)

            cw_origin = lax.rem(my + N_DEV - (h + 1), N_DEV)
            ccw_origin = lax.rem(my + h + 1, N_DEV)
            amax = chunk_gemm(cw_ref[h], cw_origin * m_per, m_per, amax)
            amax = chunk_gemm(ccw_ref[h], ccw_origin * m_per, m_per, amax)

        far = lax.rem(my + N_DEV // 2, N_DEV)
        cw[N_HOPS - 1].wait_recv()
        amax = chunk_gemm(cw_half_ref[...], far * m_per, half, amax)
        ccw[N_HOPS - 1].wait_recv()
        amax = chunk_gemm(ccw_half_ref[...], far * m_per + half, half, amax)

        amax_src[...] = jnp.full((1, 128), amax, jnp.float32)
        amax_gather[0] = amax_src[...]
        amax_rdmas = []
        for d in range(1, N_DEV):
            r = copy(amax_src, amax_gather.at[d], amax_send_sems.at[d - 1],
                     amax_recv_sems.at[d], lax.rem(my + d, N_DEV))
            r.start()
            amax_rdmas.append(r)
        for r in amax_rdmas:
            r.wait_recv()
        global_amax = jnp.max(amax_gather[...])

        scale = global_amax / 448.0
        inv = 448.0 / global_amax
        for b in range(N_DEV * 2):
            blk = out_ref[pl.ds(b * half, half), :]
            q = (blk * inv).astype(jnp.float8_e4m3fn)
            out_ref[pl.ds(b * half, half), :] = q.astype(jnp.float32) * scale

        for r in cw + ccw + amax_rdmas:
            r.wait_send()

        @functools.partial(pl.run_scoped, ack=pltpu.SemaphoreType.REGULAR)
        def _(ack):
            for d in range(1, N_DEV):
                pl.semaphore_signal(ack, inc=1,
                                    device_id=(lax.rem(my + d, N_DEV),),
                                    device_id_type=pl.DeviceIdType.MESH)
            pl.semaphore_wait(ack, N_DEV - 1)

    return pl.pallas_call(
        body,
        out_shape=jax.ShapeDtypeStruct((N_DEV * m_per, n_loc), jnp.float32),
        in_specs=[
            pl.BlockSpec(memory_space=pltpu.VMEM),
            pl.BlockSpec(memory_space=pl.ANY),
        ],
        out_specs=pl.BlockSpec(memory_space=pltpu.VMEM),
        scratch_shapes=[
            pltpu.VMEM((N_HOPS - 1, m_per, k), x.dtype),
            pltpu.VMEM((N_HOPS - 1, m_per, k), x.dtype),
            pltpu.VMEM((half, k), x.dtype),
            pltpu.VMEM((half, k), x.dtype),
            pltpu.VMEM((k, n_loc), jnp.bfloat16),
            pltpu.VMEM((k, wcols), jnp.float32),
            pltpu.VMEM((1, 128), jnp.float32),
            pltpu.VMEM((N_DEV, 1, 128), jnp.float32),
            pltpu.SemaphoreType.DMA((N_HOPS,)),
            pltpu.SemaphoreType.DMA((N_HOPS,)),
            pltpu.SemaphoreType.DMA((N_HOPS,)),
            pltpu.SemaphoreType.DMA((N_HOPS,)),
            pltpu.SemaphoreType.DMA((N_DEV - 1,)),
            pltpu.SemaphoreType.DMA((N_DEV,)),
            pltpu.SemaphoreType.DMA,
        ],
        compiler_params=pltpu.CompilerParams(
            collective_id=0, vmem_limit_bytes=64 * 1024 * 1024),
    )(x, w_mat)
